# baseline (speedup 1.0000x reference)
import functools

import jax
import jax.numpy as jnp
from jax.experimental import pallas as pl
from jax.experimental.pallas import tpu as pltpu


def _moe_mingru_body(x_ref, gw_ref, wg_ref, bg_ref, wv_ref, bv_ref,
                     wd_ref, bd_ref, out_ref, route_ref, *, B, S, E, BLK):
    d_idx = pl.program_id(0)
    e_idx = pl.program_id(1)

    xflat = x_ref[...].reshape(B * S, -1)

    @pl.when(jnp.logical_and(d_idx == 0, e_idx == 0))
    def _():
        logits = jax.lax.dot_general(
            xflat, gw_ref[...], (((1,), (1,)), ((), ())),
            preferred_element_type=jnp.float32)
        ii = jax.lax.broadcasted_iota(jnp.int32, (B * S, E), 1)
        m1 = jnp.max(logits, axis=1, keepdims=True)
        i1 = jnp.min(jnp.where(logits >= m1, ii, E), axis=1, keepdims=True)
        masked = jnp.where(ii == i1, -jnp.inf, logits)
        m2 = jnp.max(masked, axis=1, keepdims=True)
        i2 = jnp.min(jnp.where(masked >= m2, ii, E), axis=1, keepdims=True)
        z = jnp.exp(m2 - m1)
        w1 = 1.0 / (1.0 + z)
        w2 = z / (1.0 + z)
        i1f = i1.astype(jnp.float32)
        i2f = i2.astype(jnp.float32)
        route_ref[...] = jnp.concatenate(
            [w1, w2, i1f, i2f, w1, w1, w1, w1], axis=1)

    def proj(w_ref, b_ref):
        y = jax.lax.dot_general(
            xflat, w_ref[0], (((1,), (1,)), ((), ())),
            preferred_element_type=jnp.float32)
        return y + b_ref[0, pl.ds(e_idx, 1), :]

    g = proj(wg_ref, bg_ref)
    v = proj(wv_ref, bv_ref)
    d = proj(wd_ref, bd_ref)

    xs = (jax.nn.sigmoid(g) * jnp.tanh(v)).reshape(B, S, BLK)
    a = (0.001 + 0.998 * jax.nn.sigmoid(d)).reshape(B, S, BLK)

    TC = 16
    NC = S // TC
    xc = xs.reshape(B, NC, TC, BLK)
    ac = a.reshape(B, NC, TC, BLK)
    off = 1
    while off < TC:
        a_sh = jnp.concatenate(
            [jnp.ones((B, NC, off, BLK), jnp.float32),
             ac[:, :, :TC - off, :]], axis=2)
        x_sh = jnp.concatenate(
            [jnp.zeros((B, NC, off, BLK), jnp.float32),
             xc[:, :, :TC - off, :]], axis=2)
        xc = xc + ac * x_sh
        ac = ac * a_sh
        off *= 2
    xl = xc[:, :, TC - 1, :]
    al = ac[:, :, TC - 1, :]
    off = 1
    while off < NC:
        a_sh = jnp.concatenate(
            [jnp.ones((B, off, BLK), jnp.float32), al[:, :NC - off, :]],
            axis=1)
        x_sh = jnp.concatenate(
            [jnp.zeros((B, off, BLK), jnp.float32), xl[:, :NC - off, :]],
            axis=1)
        xl = xl + al * x_sh
        al = al * a_sh
        off *= 2
    carry = jnp.concatenate(
        [jnp.zeros((B, 1, BLK), jnp.float32), xl[:, :NC - 1, :]], axis=1)
    h = (xc + ac * carry[:, :, None, :]).reshape(B, S, BLK)

    r = route_ref[...]
    ef = e_idx.astype(jnp.float32)
    w_e = (r[:, 0:1] * jnp.where(r[:, 2:3] == ef, 1.0, 0.0)
           + r[:, 1:2] * jnp.where(r[:, 3:4] == ef, 1.0, 0.0))
    contrib = h * w_e.reshape(B, S, 1)

    @pl.when(e_idx == 0)
    def _():
        out_ref[...] = contrib

    @pl.when(e_idx != 0)
    def _():
        out_ref[...] = out_ref[...] + contrib


@functools.partial(jax.jit, static_argnames=("interpret",))
def kernel(x, gate_W, Wg, bg, Wv, bv, Wd, bd, interpret=False):
    B, S, D = x.shape
    E = gate_W.shape[0]
    BLK = min(128, D)
    nblk = D // BLK

    bg = bg.reshape(E, nblk, BLK).swapaxes(0, 1)
    bv = bv.reshape(E, nblk, BLK).swapaxes(0, 1)
    bd = bd.reshape(E, nblk, BLK).swapaxes(0, 1)

    body = functools.partial(_moe_mingru_body, B=B, S=S, E=E, BLK=BLK)
    return pl.pallas_call(
        body,
        grid=(nblk, E),
        in_specs=[
            pl.BlockSpec((B, S, D), lambda d, e: (0, 0, 0)),
            pl.BlockSpec((E, D), lambda d, e: (0, 0)),
            pl.BlockSpec((1, BLK, D), lambda d, e: (e, d, 0)),
            pl.BlockSpec((1, E, BLK), lambda d, e: (d, 0, 0)),
            pl.BlockSpec((1, BLK, D), lambda d, e: (e, d, 0)),
            pl.BlockSpec((1, E, BLK), lambda d, e: (d, 0, 0)),
            pl.BlockSpec((1, BLK, D), lambda d, e: (e, d, 0)),
            pl.BlockSpec((1, E, BLK), lambda d, e: (d, 0, 0)),
        ],
        out_specs=pl.BlockSpec((B, S, BLK), lambda d, e: (0, 0, d)),
        out_shape=jax.ShapeDtypeStruct((B, S, D), jnp.float32),
        scratch_shapes=[pltpu.VMEM((B * S, 8), jnp.float32)],
        interpret=interpret,
    )(x, gate_W, Wg, bg, Wv, bv, Wd, bd)


# speedup vs baseline: 1.4140x; 1.4140x over previous
import functools

import jax
import jax.numpy as jnp
from jax.experimental import pallas as pl
from jax.experimental.pallas import tpu as pltpu


def _moe_mingru_body(x_ref, gw_ref, wg_ref, bg_ref, wv_ref, bv_ref,
                     wd_ref, bd_ref, out_ref, route_ref, *, B, S, E, BLK):
    d_idx = pl.program_id(0)
    e_idx = pl.program_id(1)
    BS = B * S

    xflat = x_ref[...]

    @pl.when(jnp.logical_and(d_idx == 0, e_idx == 0))
    def _():
        logits = jax.lax.dot_general(
            xflat, gw_ref[...], (((1,), (1,)), ((), ())),
            preferred_element_type=jnp.float32)
        ii = jax.lax.broadcasted_iota(jnp.int32, (BS, E), 1)
        m1 = jnp.max(logits, axis=1, keepdims=True)
        i1 = jnp.min(jnp.where(logits >= m1, ii, E), axis=1, keepdims=True)
        masked = jnp.where(ii == i1, -jnp.inf, logits)
        m2 = jnp.max(masked, axis=1, keepdims=True)
        i2 = jnp.min(jnp.where(masked >= m2, ii, E), axis=1, keepdims=True)
        z = jnp.exp(m2 - m1)
        w1 = 1.0 / (1.0 + z)
        w2 = z / (1.0 + z)
        i1f = i1.astype(jnp.float32)
        i2f = i2.astype(jnp.float32)
        route_ref[...] = jnp.concatenate(
            [w1, w2, i1f, i2f, w1, w1, w1, w1], axis=1)

    def proj(w_ref, b_ref):
        y = jax.lax.dot_general(
            xflat, w_ref[0], (((1,), (1,)), ((), ())),
            preferred_element_type=jnp.float32)
        return y + b_ref[0, pl.ds(e_idx, 1), :]

    g = proj(wg_ref, bg_ref)
    v = proj(wv_ref, bv_ref)
    d = proj(wd_ref, bd_ref)

    xs = jax.nn.sigmoid(g) * jnp.tanh(v)
    a = 0.001 + 0.998 * jax.nn.sigmoid(d)

    rows = jax.lax.broadcasted_iota(jnp.int32, (BS, BLK), 0)
    for k in range(1, B):
        a = jnp.where(rows == k * S, 0.0, a)

    off = 1
    while off < S:
        a_sh = jnp.concatenate(
            [jnp.ones((off, BLK), jnp.float32), a[:BS - off, :]], axis=0)
        x_sh = jnp.concatenate(
            [jnp.zeros((off, BLK), jnp.float32), xs[:BS - off, :]], axis=0)
        xs = xs + a * x_sh
        if off * 2 < S:
            a = a * a_sh
        off *= 2
    h = xs

    r = route_ref[...]
    ef = e_idx.astype(jnp.float32)
    w_e = (r[:, 0:1] * jnp.where(r[:, 2:3] == ef, 1.0, 0.0)
           + r[:, 1:2] * jnp.where(r[:, 3:4] == ef, 1.0, 0.0))
    contrib = h * w_e

    @pl.when(e_idx == 0)
    def _():
        out_ref[...] = contrib

    @pl.when(e_idx != 0)
    def _():
        out_ref[...] = out_ref[...] + contrib


@functools.partial(jax.jit, static_argnames=("interpret",))
def kernel(x, gate_W, Wg, bg, Wv, bv, Wd, bd, interpret=False):
    B, S, D = x.shape
    E = gate_W.shape[0]
    BLK = min(256, D)
    nblk = D // BLK

    bg = bg.reshape(E, nblk, BLK).swapaxes(0, 1)
    bv = bv.reshape(E, nblk, BLK).swapaxes(0, 1)
    bd = bd.reshape(E, nblk, BLK).swapaxes(0, 1)
    x2 = x.reshape(B * S, D)

    body = functools.partial(_moe_mingru_body, B=B, S=S, E=E, BLK=BLK)
    out = pl.pallas_call(
        body,
        grid=(nblk, E),
        in_specs=[
            pl.BlockSpec((B * S, D), lambda d, e: (0, 0)),
            pl.BlockSpec((E, D), lambda d, e: (0, 0)),
            pl.BlockSpec((1, BLK, D), lambda d, e: (e, d, 0)),
            pl.BlockSpec((1, E, BLK), lambda d, e: (d, 0, 0)),
            pl.BlockSpec((1, BLK, D), lambda d, e: (e, d, 0)),
            pl.BlockSpec((1, E, BLK), lambda d, e: (d, 0, 0)),
            pl.BlockSpec((1, BLK, D), lambda d, e: (e, d, 0)),
            pl.BlockSpec((1, E, BLK), lambda d, e: (d, 0, 0)),
        ],
        out_specs=pl.BlockSpec((B * S, BLK), lambda d, e: (0, d)),
        out_shape=jax.ShapeDtypeStruct((B * S, D), jnp.float32),
        scratch_shapes=[pltpu.VMEM((B * S, 8), jnp.float32)],
        interpret=interpret,
    )(x2, gate_W, Wg, bg, Wv, bv, Wd, bd)
    return out.reshape(B, S, D)


# speedup vs baseline: 1.6786x; 1.1871x over previous
import functools

import jax
import jax.numpy as jnp
from jax.experimental import pallas as pl
from jax.experimental.pallas import tpu as pltpu


def _moe_mingru_body(x_ref, gw_ref, wg_ref, bg_ref, wv_ref, bv_ref,
                     wd_ref, bd_ref, out_ref, route_ref, *, B, S, E, BLK):
    d_idx = pl.program_id(0)
    e_idx = pl.program_id(1)
    BS = B * S

    xflat = x_ref[...]

    @pl.when(jnp.logical_and(d_idx == 0, e_idx == 0))
    def _():
        logits = jax.lax.dot_general(
            xflat, gw_ref[...], (((1,), (1,)), ((), ())),
            preferred_element_type=jnp.float32)
        ii = jax.lax.broadcasted_iota(jnp.int32, (BS, E), 1)
        m1 = jnp.max(logits, axis=1, keepdims=True)
        i1 = jnp.min(jnp.where(logits >= m1, ii, E), axis=1, keepdims=True)
        masked = jnp.where(ii == i1, -jnp.inf, logits)
        m2 = jnp.max(masked, axis=1, keepdims=True)
        i2 = jnp.min(jnp.where(masked >= m2, ii, E), axis=1, keepdims=True)
        z = jnp.exp(m2 - m1)
        w1 = 1.0 / (1.0 + z)
        w2 = z / (1.0 + z)
        i1f = i1.astype(jnp.float32)
        i2f = i2.astype(jnp.float32)
        rr = jax.lax.broadcasted_iota(jnp.int32, (BS, 1), 0)
        m = jnp.ones((BS, 1), jnp.float32)
        for k in range(1, B):
            m = jnp.where(rr == k * S, 0.0, m)
        route_ref[...] = jnp.concatenate(
            [w1, w2, i1f, i2f, m, m, m, m], axis=1)

    def proj(w_ref, b_ref):
        y = jax.lax.dot_general(
            xflat, w_ref[0], (((1,), (1,)), ((), ())),
            preferred_element_type=jnp.float32)
        return y + b_ref[0, pl.ds(e_idx, 1), :]

    g = proj(wg_ref, bg_ref)
    v = proj(wv_ref, bv_ref)
    d = proj(wd_ref, bd_ref)

    xs = (jax.nn.sigmoid(g) * jnp.tanh(v)).astype(jnp.bfloat16)
    a = ((0.001 + 0.998 * jax.nn.sigmoid(d)) * route_ref[:, 4:5]
         ).astype(jnp.bfloat16)

    off = 1
    while off < S:
        a_sh = jnp.concatenate(
            [jnp.ones((off, BLK), jnp.bfloat16), a[:BS - off, :]], axis=0)
        x_sh = jnp.concatenate(
            [jnp.zeros((off, BLK), jnp.bfloat16), xs[:BS - off, :]], axis=0)
        xs = xs + a * x_sh
        if off * 2 < S:
            a = a * a_sh
        off *= 2
    h = xs.astype(jnp.float32)

    r = route_ref[...]
    ef = e_idx.astype(jnp.float32)
    w_e = (r[:, 0:1] * jnp.where(r[:, 2:3] == ef, 1.0, 0.0)
           + r[:, 1:2] * jnp.where(r[:, 3:4] == ef, 1.0, 0.0))
    contrib = h * w_e

    @pl.when(e_idx == 0)
    def _():
        out_ref[...] = contrib

    @pl.when(e_idx != 0)
    def _():
        out_ref[...] = out_ref[...] + contrib


@functools.partial(jax.jit, static_argnames=("interpret",))
def kernel(x, gate_W, Wg, bg, Wv, bv, Wd, bd, interpret=False):
    B, S, D = x.shape
    E = gate_W.shape[0]
    BLK = min(256, D)
    nblk = D // BLK

    bg = bg.reshape(E, nblk, BLK).swapaxes(0, 1)
    bv = bv.reshape(E, nblk, BLK).swapaxes(0, 1)
    bd = bd.reshape(E, nblk, BLK).swapaxes(0, 1)
    x2 = x.reshape(B * S, D)

    body = functools.partial(_moe_mingru_body, B=B, S=S, E=E, BLK=BLK)
    out = pl.pallas_call(
        body,
        grid=(nblk, E),
        in_specs=[
            pl.BlockSpec((B * S, D), lambda d, e: (0, 0)),
            pl.BlockSpec((E, D), lambda d, e: (0, 0)),
            pl.BlockSpec((1, BLK, D), lambda d, e: (e, d, 0)),
            pl.BlockSpec((1, E, BLK), lambda d, e: (d, 0, 0)),
            pl.BlockSpec((1, BLK, D), lambda d, e: (e, d, 0)),
            pl.BlockSpec((1, E, BLK), lambda d, e: (d, 0, 0)),
            pl.BlockSpec((1, BLK, D), lambda d, e: (e, d, 0)),
            pl.BlockSpec((1, E, BLK), lambda d, e: (d, 0, 0)),
        ],
        out_specs=pl.BlockSpec((B * S, BLK), lambda d, e: (0, d)),
        out_shape=jax.ShapeDtypeStruct((B * S, D), jnp.float32),
        scratch_shapes=[pltpu.VMEM((B * S, 8), jnp.float32)],
        interpret=interpret,
    )(x2, gate_W, Wg, bg, Wv, bv, Wd, bd)
    return out.reshape(B, S, D)
